# baseline (speedup 1.0000x reference)
import jax
import jax.numpy as jnp
from jax import lax
from jax.experimental import pallas as pl
from jax.experimental.pallas import tpu as pltpu
from jax.experimental.pallas import tpu_sc as plsc

_VOCAB = 100000
_EMBED = 64
_BATCH = 1024
_TILE_V = 2048
_G_FULL = _VOCAB // _TILE_V
_REM = _VOCAB - _G_FULL * _TILE_V
_REM_AL = (_REM // 128) * 128
_SLIVER = _REM - _REM_AL
_NBUF = 4

_NC = 2
_NS = 16
_NW = _NC * _NS
_BPW = _BATCH // _NW


def _gather_body(table_hbm, idx_hbm, out_hbm, idx_v, rows_v, sem):
    wid = lax.axis_index("s") * _NC + lax.axis_index("c")
    base = wid * _BPW
    pltpu.sync_copy(idx_hbm.at[pl.ds(base, _BPW)], idx_v)
    pltpu.async_copy(table_hbm.at[idx_v], rows_v, sem).wait()
    pltpu.sync_copy(rows_v, out_hbm.at[pl.ds(base, _BPW)])


def _sc_gather(table, idx):
    mesh = plsc.VectorSubcoreMesh(core_axis_name="c", subcore_axis_name="s")
    k = pl.kernel(
        _gather_body,
        mesh=mesh,
        out_type=jax.ShapeDtypeStruct((_BATCH, _EMBED), jnp.float32),
        scratch_types=[
            pltpu.VMEM((_BPW,), jnp.int32),
            pltpu.VMEM((_BPW, _EMBED), jnp.float32),
            pltpu.SemaphoreType.DMA,
        ],
        compiler_params=pltpu.CompilerParams(use_tc_tiling_on_sc=False),
    )
    return k(table, idx)


def _mm_body(e_ref, w_ref, b_ref, out_hbm, acc, sems, sem_tail):
    i = pl.program_id(0)
    buf = lax.rem(i, _NBUF)

    @pl.when(i >= _NBUF)
    def _drain_prev():
        pltpu.make_async_copy(
            acc.at[buf],
            out_hbm.at[:, pl.ds((i - _NBUF) * _TILE_V, _TILE_V)],
            sems.at[buf],
        ).wait()

    acc[buf, :, :] = lax.dot_general(
        e_ref[...], w_ref[...],
        (((1,), (1,)), ((), ())),
        preferred_element_type=jnp.float32,
    ) + b_ref[...]

    for _b in range(_NBUF):
        @pl.when((buf == _b) & (i < _G_FULL))
        def _issue(_b=_b):
            pltpu.make_async_copy(
                acc.at[_b],
                out_hbm.at[:, pl.ds(i * _TILE_V, _TILE_V)],
                sems.at[_b],
            ).start(priority=_b % 2)

    @pl.when(i == _G_FULL)
    def _tail_and_drain():
        pltpu.make_async_copy(
            acc.at[buf, :, :_REM_AL],
            out_hbm.at[:, pl.ds(_G_FULL * _TILE_V, _REM_AL)],
            sem_tail,
        ).start()
        for k in range(1, _NBUF):
            b = (_G_FULL + k) % _NBUF
            pltpu.make_async_copy(
                acc.at[b],
                out_hbm.at[:, pl.ds((_G_FULL - _NBUF + k) * _TILE_V, _TILE_V)],
                sems.at[b],
            ).wait()
        pltpu.make_async_copy(
            acc.at[buf, :, :_REM_AL],
            out_hbm.at[:, pl.ds(_G_FULL * _TILE_V, _REM_AL)],
            sem_tail,
        ).wait()


def _decode(e, dec_W, dec_b2d):
    return pl.pallas_call(
        _mm_body,
        grid=(_G_FULL + 1,),
        in_specs=[
            pl.BlockSpec((_BATCH, _EMBED), lambda i: (0, 0)),
            pl.BlockSpec((_TILE_V, _EMBED), lambda i: (i, 0)),
            pl.BlockSpec((1, _TILE_V), lambda i: (0, i)),
        ],
        out_specs=pl.BlockSpec(memory_space=pltpu.MemorySpace.HBM),
        out_shape=jax.ShapeDtypeStruct((_BATCH, _VOCAB), jnp.float32),
        scratch_shapes=[
            pltpu.VMEM((_NBUF, _BATCH, _TILE_V), jnp.float32),
            pltpu.SemaphoreType.DMA((_NBUF,)),
            pltpu.SemaphoreType.DMA,
        ],
    )(e, dec_W, dec_b2d)


def _sliver_body(big_ref, e_ref, w_ref, b_ref, out_ref):
    del big_ref
    out_ref[...] = lax.dot_general(
        e_ref[...], w_ref[...],
        (((1,), (1,)), ((), ())),
        preferred_element_type=jnp.float32,
    ) + b_ref[...]


def _write_sliver(big, e, dec_W, dec_b2d):
    blk = _VOCAB // 128
    return pl.pallas_call(
        _sliver_body,
        grid=(1,),
        in_specs=[
            pl.BlockSpec(memory_space=pltpu.MemorySpace.HBM),
            pl.BlockSpec((_BATCH, _EMBED), lambda i: (0, 0)),
            pl.BlockSpec((128, _EMBED), lambda i: (blk, 0)),
            pl.BlockSpec((1, 128), lambda i: (0, blk)),
        ],
        out_specs=pl.BlockSpec((_BATCH, 128), lambda i: (0, blk)),
        out_shape=jax.ShapeDtypeStruct((_BATCH, _VOCAB), jnp.float32),
        input_output_aliases={0: 0},
    )(big, e, dec_W, dec_b2d)


def kernel(center_words, emb_table, dec_W, dec_b):
    idx = center_words.astype(jnp.int32)
    e = _sc_gather(emb_table, idx)
    b2d = dec_b.reshape(1, _VOCAB)
    big = _decode(e, dec_W, b2d)
    return _write_sliver(big, e, dec_W, b2d)


# speedup vs baseline: 2.3420x; 2.3420x over previous
import jax
import jax.numpy as jnp
from jax import lax
from jax.experimental import pallas as pl
from jax.experimental.pallas import tpu as pltpu
from jax.experimental.pallas import tpu_sc as plsc

_VOCAB = 100000
_EMBED = 64
_BATCH = 1024
_TILE_V = 2048

_NC = 2
_NS = 16
_NW = _NC * _NS
_BPW = _BATCH // _NW


def _gather_body(table_hbm, idx_hbm, out_hbm, idx_v, rows_v, sem):
    wid = lax.axis_index("s") * _NC + lax.axis_index("c")
    base = wid * _BPW
    pltpu.sync_copy(idx_hbm.at[pl.ds(base, _BPW)], idx_v)
    pltpu.async_copy(table_hbm.at[idx_v], rows_v, sem).wait()
    pltpu.sync_copy(rows_v, out_hbm.at[pl.ds(base, _BPW)])


def _sc_gather(table, idx):
    mesh = plsc.VectorSubcoreMesh(core_axis_name="c", subcore_axis_name="s")
    k = pl.kernel(
        _gather_body,
        mesh=mesh,
        out_type=jax.ShapeDtypeStruct((_BATCH, _EMBED), jnp.float32),
        scratch_types=[
            pltpu.VMEM((_BPW,), jnp.int32),
            pltpu.VMEM((_BPW, _EMBED), jnp.float32),
            pltpu.SemaphoreType.DMA,
        ],
        compiler_params=pltpu.CompilerParams(use_tc_tiling_on_sc=False),
    )
    return k(table, idx)


def _mm_body(w_ref, e_ref, b_ref, out_ref):
    out_ref[...] = lax.dot_general(
        w_ref[...], e_ref[...],
        (((1,), (1,)), ((), ())),
        preferred_element_type=jnp.float32,
    ) + b_ref[...].reshape(_TILE_V, 1)


def _decode_t(dec_W, e, dec_b2d):
    return pl.pallas_call(
        _mm_body,
        grid=(pl.cdiv(_VOCAB, _TILE_V),),
        in_specs=[
            pl.BlockSpec((_TILE_V, _EMBED), lambda i: (i, 0)),
            pl.BlockSpec((_BATCH, _EMBED), lambda i: (0, 0)),
            pl.BlockSpec((1, _TILE_V), lambda i: (0, i)),
        ],
        out_specs=pl.BlockSpec((_TILE_V, _BATCH), lambda i: (i, 0)),
        out_shape=jax.ShapeDtypeStruct((_VOCAB, _BATCH), jnp.float32),
    )(dec_W, e, dec_b2d)


def kernel(center_words, emb_table, dec_W, dec_b):
    idx = center_words.astype(jnp.int32)
    e = _sc_gather(emb_table, idx)
    out_t = _decode_t(dec_W, e, dec_b.reshape(1, _VOCAB))
    return out_t.T


# speedup vs baseline: 2.3814x; 1.0168x over previous
import jax
import jax.numpy as jnp
from jax import lax
from jax.experimental import pallas as pl
from jax.experimental.pallas import tpu as pltpu
from jax.experimental.pallas import tpu_sc as plsc

_VOCAB = 100000
_EMBED = 64
_BATCH = 1024
_TILE_V = 4096

_NC = 2
_NS = 16
_NW = _NC * _NS
_BPW = _BATCH // _NW


def _gather_body(table_hbm, idx_hbm, out_hbm, idx_v, rows_v, sem):
    wid = lax.axis_index("s") * _NC + lax.axis_index("c")
    base = wid * _BPW
    pltpu.sync_copy(idx_hbm.at[pl.ds(base, _BPW)], idx_v)
    pltpu.async_copy(table_hbm.at[idx_v], rows_v, sem).wait()
    pltpu.sync_copy(rows_v, out_hbm.at[pl.ds(base, _BPW)])


def _sc_gather(table, idx):
    mesh = plsc.VectorSubcoreMesh(core_axis_name="c", subcore_axis_name="s")
    k = pl.kernel(
        _gather_body,
        mesh=mesh,
        out_type=jax.ShapeDtypeStruct((_BATCH, _EMBED), jnp.float32),
        scratch_types=[
            pltpu.VMEM((_BPW,), jnp.int32),
            pltpu.VMEM((_BPW, _EMBED), jnp.float32),
            pltpu.SemaphoreType.DMA,
        ],
        compiler_params=pltpu.CompilerParams(use_tc_tiling_on_sc=False),
    )
    return k(table, idx)


def _mm_body(w_ref, e_ref, b_ref, out_ref):
    out_ref[...] = lax.dot_general(
        w_ref[...], e_ref[...],
        (((1,), (1,)), ((), ())),
        preferred_element_type=jnp.float32,
    ) + b_ref[...].reshape(_TILE_V, 1)


def _decode_t(dec_W, e, dec_b2d):
    return pl.pallas_call(
        _mm_body,
        grid=(pl.cdiv(_VOCAB, _TILE_V),),
        in_specs=[
            pl.BlockSpec((_TILE_V, _EMBED), lambda i: (i, 0)),
            pl.BlockSpec((_BATCH, _EMBED), lambda i: (0, 0)),
            pl.BlockSpec((1, _TILE_V), lambda i: (0, i)),
        ],
        out_specs=pl.BlockSpec((_TILE_V, _BATCH), lambda i: (i, 0)),
        out_shape=jax.ShapeDtypeStruct((_VOCAB, _BATCH), jnp.float32),
    )(dec_W, e, dec_b2d)


def kernel(center_words, emb_table, dec_W, dec_b):
    idx = center_words.astype(jnp.int32)
    e = _sc_gather(emb_table, idx)
    out_t = _decode_t(dec_W, e, dec_b.reshape(1, _VOCAB))
    return out_t.T


# speedup vs baseline: 2.3851x; 1.0015x over previous
import jax
import jax.numpy as jnp
from jax import lax
from jax.experimental import pallas as pl
from jax.experimental.pallas import tpu as pltpu
from jax.experimental.pallas import tpu_sc as plsc

_VOCAB = 100000
_EMBED = 64
_BATCH = 1024
_TILE_V = 4096

_NC = 2
_NS = 16
_NW = _NC * _NS
_BPW = _BATCH // _NW


def _gather_body(table_hbm, idx_hbm, out_hbm, idx_v, rows_v, sem):
    wid = lax.axis_index("s") * _NC + lax.axis_index("c")
    base = wid * _BPW
    pltpu.sync_copy(idx_hbm.at[pl.ds(base, _BPW)], idx_v)
    pltpu.async_copy(table_hbm.at[idx_v], rows_v, sem).wait()
    pltpu.sync_copy(rows_v, out_hbm.at[pl.ds(base, _BPW)])


def _sc_gather(table, idx):
    mesh = plsc.VectorSubcoreMesh(core_axis_name="c", subcore_axis_name="s")
    k = pl.kernel(
        _gather_body,
        mesh=mesh,
        out_type=jax.ShapeDtypeStruct((_BATCH, _EMBED), jnp.float32),
        scratch_types=[
            pltpu.VMEM((_BPW,), jnp.int32),
            pltpu.VMEM((_BPW, _EMBED), jnp.float32),
            pltpu.SemaphoreType.DMA,
        ],
        compiler_params=pltpu.CompilerParams(use_tc_tiling_on_sc=False),
    )
    return k(table, idx)


def _mm_body(w_ref, e_ref, b_ref, out_ref):
    out_ref[...] = lax.dot_general(
        w_ref[...], e_ref[...],
        (((1,), (1,)), ((), ())),
        preferred_element_type=jnp.float32,
    ) + b_ref[...].reshape(_TILE_V, 1)


def _decode_t(dec_W, e, dec_b2d):
    return pl.pallas_call(
        _mm_body,
        grid=(pl.cdiv(_VOCAB, _TILE_V),),
        in_specs=[
            pl.BlockSpec((_TILE_V, _EMBED), lambda i: (i, 0)),
            pl.BlockSpec((_BATCH, _EMBED), lambda i: (0, 0)),
            pl.BlockSpec((1, _TILE_V), lambda i: (0, i)),
        ],
        out_specs=pl.BlockSpec((_TILE_V, _BATCH), lambda i: (i, 0),
                               pipeline_mode=pl.Buffered(buffer_count=2)),
        out_shape=jax.ShapeDtypeStruct((_VOCAB, _BATCH), jnp.float32),
    )(dec_W, e, dec_b2d)


def kernel(center_words, emb_table, dec_W, dec_b):
    idx = center_words.astype(jnp.int32)
    e = _sc_gather(emb_table, idx)
    out_t = _decode_t(dec_W, e, dec_b.reshape(1, _VOCAB))
    return out_t.T
